# baseline (speedup 1.0000x reference)
import functools

import jax
import jax.numpy as jnp
from jax import lax
from jax.experimental import pallas as pl
from jax.experimental.pallas import tpu as pltpu
from jax.experimental.pallas import tpu_sc as plsc

N = 10000
E = 320000
D = 128
H1 = 8
NREL = 401
UW = 144

NC = 2
NS = 16
LANES = 16


_EMB_B = 10240
_EMB_PER_W = _EMB_B // (NC * NS)


def _emb_body(table, idx, out, idx_v, rows_v, sem):
    wid = lax.axis_index("s") * NC + lax.axis_index("c")
    base = wid * _EMB_PER_W
    pltpu.sync_copy(idx.at[pl.ds(base, _EMB_PER_W)], idx_v)
    pltpu.async_copy(table.at[idx_v], rows_v, sem).wait()
    pltpu.sync_copy(rows_v, out.at[pl.ds(base, _EMB_PER_W)])


def _emb_gather(table, idx_padded):
    mesh = plsc.VectorSubcoreMesh(core_axis_name="c", subcore_axis_name="s")
    return pl.kernel(
        _emb_body,
        out_type=jax.ShapeDtypeStruct((_EMB_B, D), jnp.float32),
        mesh=mesh,
        scratch_types=[
            pltpu.VMEM((_EMB_PER_W,), jnp.int32),
            pltpu.VMEM((_EMB_PER_W, D), jnp.float32),
            pltpu.SemaphoreType.DMA,
        ],
        compiler_params=pltpu.CompilerParams(
            needs_layout_passes=False, use_tc_tiling_on_sc=False),
    )(table, idx_padded)


CHUNK = 16
EPT = E // (NC * NS)
NCHUNK = EPT // CHUNK
NP = 10240
ROWS_PER_S = NP // NS


def _edge_body(xtab, atabf, idx3, zcol, z128, U, Dn,
               ixA, ixB, xi0, xj0, xi1, xj1, msg_v,
               pt_v, ee_v, atb_v, dn_v, U_sh,
               s1, s2, s4, s5, si):
    c = lax.axis_index("c")
    s = lax.axis_index("s")
    tid = c * NS + s
    iota = jnp.arange(LANES, dtype=jnp.int32)

    r0 = s * ROWS_PER_S
    pltpu.sync_copy(z128.at[pl.ds(r0, ROWS_PER_S)],
                    U_sh.at[pl.ds(r0, ROWS_PER_S)])
    pltpu.sync_copy(zcol, dn_v)
    pltpu.sync_copy(atabf, atb_v)
    plsc.subcore_barrier()

    rbase = tid * NCHUNK

    def issue(ix, xi_v, xj_v, sa, sb):
        cp1 = pltpu.async_copy(xtab.at[ix.at[1]], xi_v, sa)
        cp2 = pltpu.async_copy(xtab.at[ix.at[0]], xj_v, sb)
        return cp1, cp2

    def compute_scatter(ix, xi_v, xj_v):
        relvec = ix[2, :]
        dvec = ix[1, :]
        for h in range(4):
            f0 = h * 4
            accs = [None] * 4
            for kk in range(4):
                for e in range(4):
                    alo, ahi = plsc.unpack(
                        atb_v[relvec[f0 + e], pl.ds(kk * 32, 32)],
                        format=plsc.PackFormat.INTERLEAVED)
                    ilo, ihi = plsc.unpack(
                        xi_v[f0 + e, pl.ds(kk * 32, 32)],
                        format=plsc.PackFormat.INTERLEAVED)
                    jlo, jhi = plsc.unpack(
                        xj_v[f0 + e, pl.ds(kk * 32, 32)],
                        format=plsc.PackFormat.INTERLEAVED)
                    p = ilo * alo * jlo + ihi * ahi * jhi
                    accs[e] = p if kk == 0 else accs[e] + p
            for e in range(4):
                plsc.store_scatter(
                    pt_v, [iota, jnp.full((LANES,), f0 + e, jnp.int32)],
                    accs[e])
        ps = [pt_v[k, :] for k in range(16)]
        while len(ps) > 1:
            ps = [ps[i] + ps[i + 1] for i in range(0, len(ps), 2)]
        alpha = ps[0]
        alpha = jnp.where(alpha > 0, alpha, 0.2 * alpha)
        ee = jnp.exp(alpha)
        ee_v[:] = ee
        plsc.addupdate_scatter(dn_v, [dvec], ee)
        for h in range(4):
            f0 = h * 4
            ebs = [plsc.load_gather(
                ee_v, [jnp.full((LANES,), f0 + e, jnp.int32)])
                for e in range(4)]
            for kk in range(4):
                for e in range(4):
                    jlo, jhi = plsc.unpack(
                        xj_v[f0 + e, pl.ds(kk * 32, 32)],
                        format=plsc.PackFormat.INTERLEAVED)
                    msg_v[f0 + e, pl.ds(kk * 32, 16)] = ebs[e] * jlo
                    msg_v[f0 + e, pl.ds(kk * 32 + 16, 16)] = ebs[e] * jhi
        pltpu.sync_copy(msg_v, U_sh.at[ix.at[1]], add=True)

    def wait_set(ix, xi_v, xj_v, sa, sb):
        pltpu.make_async_copy(xtab.at[ix.at[1]], xi_v, sa).wait()
        pltpu.make_async_copy(xtab.at[ix.at[0]], xj_v, sb).wait()

    pltpu.sync_copy(idx3.at[rbase], ixA)
    pltpu.sync_copy(idx3.at[rbase + 1], ixB)
    issue(ixA, xi0, xj0, s1, s2)
    issue(ixB, xi1, xj1, s4, s5)

    def body2(k, _):
        j2 = 2 * k + 2
        j3 = 2 * k + 3
        wait_set(ixA, xi0, xj0, s1, s2)
        compute_scatter(ixA, xi0, xj0)

        @pl.when(j2 < NCHUNK)
        def _():
            pltpu.sync_copy(idx3.at[rbase + j2], ixA)
            issue(ixA, xi0, xj0, s1, s2)

        wait_set(ixB, xi1, xj1, s4, s5)
        compute_scatter(ixB, xi1, xj1)

        @pl.when(j3 < NCHUNK)
        def _():
            pltpu.sync_copy(idx3.at[rbase + j3], ixB)
            issue(ixB, xi1, xj1, s4, s5)

        return ()

    lax.fori_loop(0, NCHUNK // 2, body2, ())
    wait_set(ixA, xi0, xj0, s1, s2)
    compute_scatter(ixA, xi0, xj0)

    plsc.subcore_barrier()
    pltpu.sync_copy(U_sh.at[pl.ds(r0, ROWS_PER_S)],
                    U.at[pl.ds(c * NP + r0, ROWS_PER_S)])
    pltpu.sync_copy(dn_v, Dn.at[c, s])


def _edge_pass(xtab, atabf, idx3, zcol, z128):
    mesh = plsc.VectorSubcoreMesh(core_axis_name="c", subcore_axis_name="s")
    return pl.kernel(
        _edge_body,
        out_type=[jax.ShapeDtypeStruct((2 * NP, D), jnp.float32),
                  jax.ShapeDtypeStruct((NC, NS, NP), jnp.float32)],
        mesh=mesh,
        scratch_types=[
            pltpu.VMEM((3, CHUNK), jnp.int32),
            pltpu.VMEM((3, CHUNK), jnp.int32),
            pltpu.VMEM((CHUNK, D), jnp.bfloat16),
            pltpu.VMEM((CHUNK, D), jnp.bfloat16),
            pltpu.VMEM((CHUNK, D), jnp.bfloat16),
            pltpu.VMEM((CHUNK, D), jnp.bfloat16),
            pltpu.VMEM((CHUNK, D), jnp.float32),
            pltpu.VMEM((LANES, LANES), jnp.float32),
            pltpu.VMEM((LANES,), jnp.float32),
            pltpu.VMEM((NREL, D), jnp.bfloat16),
            pltpu.VMEM((NP,), jnp.float32),
            pltpu.VMEM_SHARED((NP, D), jnp.float32),
            pltpu.SemaphoreType.DMA,
            pltpu.SemaphoreType.DMA,
            pltpu.SemaphoreType.DMA,
            pltpu.SemaphoreType.DMA,
            pltpu.SemaphoreType.DMA,
        ],
        compiler_params=pltpu.CompilerParams(
            needs_layout_passes=False, use_tc_tiling_on_sc=False),
    )(xtab, atabf, idx3, zcol, z128)


_BLK1 = 2000


def _mm1_body(x0_ref, w_ref, aself_ref, x1_ref, x1b_ref, ui_ref):
    x0 = x0_ref[...]
    w = w_ref[0]
    y = jnp.dot(x0, w, preferred_element_type=jnp.float32)
    a = aself_ref[0]
    alpha = jnp.sum(y * y * a, axis=-1)
    alpha = jnp.where(alpha > 0, alpha, 0.2 * alpha)
    ee = jnp.exp(alpha)[:, None]
    x1_ref[0] = y
    x1b_ref[0] = y.reshape(-1, 4, 2, 16).swapaxes(2, 3).reshape(
        -1, D).astype(jnp.bfloat16)
    ui_ref[0, :, :D] = ee * y
    lane16 = (lax.broadcasted_iota(jnp.int32, (1, 16), 1) == 0)
    ui_ref[0, :, D:UW] = ee * lane16.astype(jnp.float32)


def _mm1(x0, W1r, a1self):
    nb = N // _BLK1
    return pl.pallas_call(
        _mm1_body,
        grid=(H1, nb),
        in_specs=[
            pl.BlockSpec((_BLK1, D), lambda h, i: (i, 0)),
            pl.BlockSpec((1, D, D), lambda h, i: (h, 0, 0)),
            pl.BlockSpec((1, 1, D), lambda h, i: (h, 0, 0)),
        ],
        out_specs=[
            pl.BlockSpec((1, _BLK1, D), lambda h, i: (h, i, 0)),
            pl.BlockSpec((1, _BLK1, D), lambda h, i: (h, i, 0)),
            pl.BlockSpec((1, _BLK1, UW), lambda h, i: (h, i, 0)),
        ],
        out_shape=[
            jax.ShapeDtypeStruct((H1, N, D), jnp.float32),
            jax.ShapeDtypeStruct((H1, N, D), jnp.bfloat16),
            jax.ShapeDtypeStruct((H1, N, UW), jnp.float32),
        ],
    )(x0, W1r, a1self)


_BLK2 = 400


def _mid_body(ua_ref, ub_ref, dn_ref, ui_ref, b1_ref, w2_ref, a2_ref,
              x2_ref, x2b_ref, ui2_ref):
    x2 = jnp.zeros((_BLK2, D), dtype=jnp.float32)
    for h in range(H1):
        num = ua_ref[h] + ub_ref[h] + ui_ref[h][:, :D]
        den = jnp.sum(dn_ref[h], axis=-1, keepdims=True) \
            + ui_ref[h][:, D:D + 1]
        o = num / (den + 1e-16) + b1_ref[h][None, :]
        hh = jnp.where(o > 0, o, 0.01 * o)
        x2 = x2 + jnp.dot(hh, w2_ref[h], preferred_element_type=jnp.float32)
    a2 = a2_ref[...]
    alpha = jnp.sum(x2 * x2 * a2, axis=-1)
    alpha = jnp.where(alpha > 0, alpha, 0.2 * alpha)
    ee = jnp.exp(alpha)[:, None]
    x2_ref[...] = x2
    x2b_ref[...] = x2.reshape(-1, 4, 2, 16).swapaxes(2, 3).reshape(
        -1, D).astype(jnp.bfloat16)
    ui2_ref[:, :D] = ee * x2
    lane16 = (lax.broadcasted_iota(jnp.int32, (1, 16), 1) == 0)
    ui2_ref[:, D:UW] = ee * lane16.astype(jnp.float32)


def _mid(Ua, Ub, dn1t, uinit1, b1r, W2r, a2self):
    nb = N // _BLK2
    return pl.pallas_call(
        _mid_body,
        grid=(nb,),
        in_specs=[
            pl.BlockSpec((H1, _BLK2, D), lambda i: (0, i, 0)),
            pl.BlockSpec((H1, _BLK2, D), lambda i: (0, i, 0)),
            pl.BlockSpec((H1, _BLK2, NC * NS), lambda i: (0, i, 0)),
            pl.BlockSpec((H1, _BLK2, UW), lambda i: (0, i, 0)),
            pl.BlockSpec((H1, D), lambda i: (0, 0)),
            pl.BlockSpec((H1, D, D), lambda i: (0, 0, 0)),
            pl.BlockSpec((1, D), lambda i: (0, 0)),
        ],
        out_specs=[
            pl.BlockSpec((_BLK2, D), lambda i: (i, 0)),
            pl.BlockSpec((_BLK2, D), lambda i: (i, 0)),
            pl.BlockSpec((_BLK2, UW), lambda i: (i, 0)),
        ],
        out_shape=[
            jax.ShapeDtypeStruct((N, D), jnp.float32),
            jax.ShapeDtypeStruct((N, D), jnp.bfloat16),
            jax.ShapeDtypeStruct((N, UW), jnp.float32),
        ],
    )(Ua, Ub, dn1t, uinit1, b1r, W2r, a2self)


_BLK3 = 2000


def _fin_body(ua_ref, ub_ref, dn_ref, ui_ref, b2_ref, out_ref):
    num = ua_ref[...] + ub_ref[...] + ui_ref[:, :D]
    den = jnp.sum(dn_ref[...], axis=-1, keepdims=True) + ui_ref[:, D:D + 1]
    out_ref[...] = num / (den + 1e-16) + b2_ref[...]


def _fin(Ua, Ub, dn2t, uinit2, b2):
    nb = N // _BLK3
    return pl.pallas_call(
        _fin_body,
        grid=(nb,),
        in_specs=[
            pl.BlockSpec((_BLK3, D), lambda i: (i, 0)),
            pl.BlockSpec((_BLK3, D), lambda i: (i, 0)),
            pl.BlockSpec((_BLK3, NC * NS), lambda i: (i, 0)),
            pl.BlockSpec((_BLK3, UW), lambda i: (i, 0)),
            pl.BlockSpec((1, D), lambda i: (0, 0)),
        ],
        out_specs=pl.BlockSpec((_BLK3, D), lambda i: (i, 0)),
        out_shape=jax.ShapeDtypeStruct((N, D), jnp.float32),
    )(Ua, Ub, dn2t, uinit2, b2)


def kernel(entity, edge_index, edge_attr, emb_table, W1, att1, b1, W2, att2,
           b2):
    src = edge_index[0].reshape(NC * NS, NCHUNK, CHUNK)
    dst = edge_index[1].reshape(NC * NS, NCHUNK, CHUNK)
    ea = edge_attr.reshape(NC * NS, NCHUNK, CHUNK)
    idx3 = jnp.stack([src, dst, ea], axis=2).reshape(
        NC * NS * NCHUNK, 3, CHUNK)
    zcol = jnp.zeros((NP,), dtype=jnp.float32)
    z128 = jnp.zeros((NP, D), dtype=jnp.float32)

    def interleave_bf16(a):
        R = a.shape[0]
        return a.reshape(R, 4, 2, 16).transpose(0, 1, 3, 2).reshape(
            R, D).astype(jnp.bfloat16)

    idx_pad = jnp.pad(entity, (0, _EMB_B - N))
    x0 = _emb_gather(emb_table, idx_pad)[:N]

    W1r = W1.reshape(D, H1, D).transpose(1, 0, 2)
    a1self = att1[NREL - 1].reshape(H1, 1, D)
    x1h, x1b, uinit1 = _mm1(x0, W1r, a1self)

    att1h = att1.transpose(1, 0, 2)
    res1 = [_edge_pass(x1b[h], interleave_bf16(att1h[h]), idx3, zcol, z128)
            for h in range(H1)]
    Ua = jnp.stack([u[:N] for u, _ in res1])
    Ub = jnp.stack([u[NP:NP + N] for u, _ in res1])
    dn1t = jnp.stack([dn.reshape(NC * NS, NP)[:, :N].T for _, dn in res1])

    b1r = b1.reshape(H1, D)
    W2r = W2.reshape(H1, D, D)
    a2self = att2[NREL - 1]
    x2, x2b, uinit2 = _mid(Ua, Ub, dn1t, uinit1, b1r, W2r, a2self)

    U2, dn2 = _edge_pass(x2b, interleave_bf16(att2[:, 0, :]), idx3, zcol,
                         z128)
    dn2t = dn2.reshape(NC * NS, NP)[:, :N].T
    out = _fin(U2[:N], U2[NP:NP + N], dn2t, uinit2, b2.reshape(1, D))
    return out


# speedup vs baseline: 1.2102x; 1.2102x over previous
import functools

import jax
import jax.numpy as jnp
from jax import lax
from jax.experimental import pallas as pl
from jax.experimental.pallas import tpu as pltpu
from jax.experimental.pallas import tpu_sc as plsc

N = 10000
E = 320000
D = 128
H1 = 8
NREL = 401
UW = 144

NC = 2
NS = 16
LANES = 16


_EMB_B = 10240
_EMB_PER_W = _EMB_B // (NC * NS)


def _emb_body(table, idx, out, idx_v, rows_v, sem):
    wid = lax.axis_index("s") * NC + lax.axis_index("c")
    base = wid * _EMB_PER_W
    pltpu.sync_copy(idx.at[pl.ds(base, _EMB_PER_W)], idx_v)
    pltpu.async_copy(table.at[idx_v], rows_v, sem).wait()
    pltpu.sync_copy(rows_v, out.at[pl.ds(base, _EMB_PER_W)])


def _emb_gather(table, idx_padded):
    mesh = plsc.VectorSubcoreMesh(core_axis_name="c", subcore_axis_name="s")
    return pl.kernel(
        _emb_body,
        out_type=jax.ShapeDtypeStruct((_EMB_B, D), jnp.float32),
        mesh=mesh,
        scratch_types=[
            pltpu.VMEM((_EMB_PER_W,), jnp.int32),
            pltpu.VMEM((_EMB_PER_W, D), jnp.float32),
            pltpu.SemaphoreType.DMA,
        ],
        compiler_params=pltpu.CompilerParams(
            needs_layout_passes=False, use_tc_tiling_on_sc=False),
    )(table, idx_padded)


CHUNK = 16
EPT = E // (NC * NS)
NCHUNK = EPT // CHUNK
NP = 10240
ROWS_PER_S = NP // NS


def _edge_body(xtab, atabf, idx3, zcol, z128, U, Dn,
               ixA, ixB, xi0, xj0, xi1, xj1,
               pt_v, ee_v, atb_v, dn_v, U_sh,
               s1, s2, s4, s5, si):
    c = lax.axis_index("c")
    s = lax.axis_index("s")
    tid = c * NS + s
    iota = jnp.arange(LANES, dtype=jnp.int32)

    r0 = s * ROWS_PER_S
    pltpu.sync_copy(z128.at[pl.ds(r0, ROWS_PER_S)],
                    U_sh.at[pl.ds(r0, ROWS_PER_S)])
    pltpu.sync_copy(zcol, dn_v)
    pltpu.sync_copy(atabf, atb_v)
    plsc.subcore_barrier()

    rbase = tid * NCHUNK

    def issue(ix, xi_v, xj_v, sa, sb):
        cp1 = pltpu.async_copy(xtab.at[ix.at[1]], xi_v, sa)
        cp2 = pltpu.async_copy(xtab.at[ix.at[0]], xj_v, sb)
        return cp1, cp2

    def compute_scatter(ix, xi_v, xj_v):
        relvec = ix[2, :]
        dvec = ix[1, :]
        for h in range(4):
            f0 = h * 4
            accs = [None] * 4
            for kk in range(4):
                ats = []
                for e in range(4):
                    ab = atb_v[relvec[f0 + e], pl.ds(kk * 32, 32)]
                    ats.append(plsc.unpack(
                        ab, format=plsc.PackFormat.INTERLEAVED))
                for e in range(4):
                    lo, hi = ats[e]
                    p = xi_v[f0 + e, pl.ds(kk * 32, 16)] * lo \
                        * xj_v[f0 + e, pl.ds(kk * 32, 16)] \
                        + xi_v[f0 + e, pl.ds(kk * 32 + 16, 16)] * hi \
                        * xj_v[f0 + e, pl.ds(kk * 32 + 16, 16)]
                    accs[e] = p if kk == 0 else accs[e] + p
            for e in range(4):
                plsc.store_scatter(
                    pt_v, [iota, jnp.full((LANES,), f0 + e, jnp.int32)],
                    accs[e])
        ps = [pt_v[k, :] for k in range(16)]
        while len(ps) > 1:
            ps = [ps[i] + ps[i + 1] for i in range(0, len(ps), 2)]
        alpha = ps[0]
        alpha = jnp.where(alpha > 0, alpha, 0.2 * alpha)
        ee = jnp.exp(alpha)
        ee_v[:] = ee
        plsc.addupdate_scatter(dn_v, [dvec], ee)
        for h in range(4):
            f0 = h * 4
            ebs = [plsc.load_gather(
                ee_v, [jnp.full((LANES,), f0 + e, jnp.int32)])
                for e in range(4)]
            for k in range(8):
                for e in range(4):
                    xi_v[f0 + e, pl.ds(k * 16, 16)] = \
                        ebs[e] * xj_v[f0 + e, pl.ds(k * 16, 16)]
        pltpu.sync_copy(xi_v, U_sh.at[ix.at[1]], add=True)

    def wait_set(ix, xi_v, xj_v, sa, sb):
        pltpu.make_async_copy(xtab.at[ix.at[1]], xi_v, sa).wait()
        pltpu.make_async_copy(xtab.at[ix.at[0]], xj_v, sb).wait()

    pltpu.sync_copy(idx3.at[rbase], ixA)
    pltpu.sync_copy(idx3.at[rbase + 1], ixB)
    issue(ixA, xi0, xj0, s1, s2)
    issue(ixB, xi1, xj1, s4, s5)

    def body2(k, _):
        j2 = 2 * k + 2
        j3 = 2 * k + 3
        wait_set(ixA, xi0, xj0, s1, s2)
        compute_scatter(ixA, xi0, xj0)

        @pl.when(j2 < NCHUNK)
        def _():
            pltpu.sync_copy(idx3.at[rbase + j2], ixA)
            issue(ixA, xi0, xj0, s1, s2)

        wait_set(ixB, xi1, xj1, s4, s5)
        compute_scatter(ixB, xi1, xj1)

        @pl.when(j3 < NCHUNK)
        def _():
            pltpu.sync_copy(idx3.at[rbase + j3], ixB)
            issue(ixB, xi1, xj1, s4, s5)

        return ()

    lax.fori_loop(0, NCHUNK // 2, body2, ())
    wait_set(ixA, xi0, xj0, s1, s2)
    compute_scatter(ixA, xi0, xj0)

    plsc.subcore_barrier()
    pltpu.sync_copy(U_sh.at[pl.ds(r0, ROWS_PER_S)],
                    U.at[pl.ds(c * NP + r0, ROWS_PER_S)])
    pltpu.sync_copy(dn_v, Dn.at[c, s])


def _edge_pass(xtab, atabf, idx3, zcol, z128):
    mesh = plsc.VectorSubcoreMesh(core_axis_name="c", subcore_axis_name="s")
    return pl.kernel(
        _edge_body,
        out_type=[jax.ShapeDtypeStruct((2 * NP, D), jnp.float32),
                  jax.ShapeDtypeStruct((NC, NS, NP), jnp.float32)],
        mesh=mesh,
        scratch_types=[
            pltpu.VMEM((3, CHUNK), jnp.int32),
            pltpu.VMEM((3, CHUNK), jnp.int32),
            pltpu.VMEM((CHUNK, D), jnp.float32),
            pltpu.VMEM((CHUNK, D), jnp.float32),
            pltpu.VMEM((CHUNK, D), jnp.float32),
            pltpu.VMEM((CHUNK, D), jnp.float32),
            pltpu.VMEM((LANES, LANES), jnp.float32),
            pltpu.VMEM((LANES,), jnp.float32),
            pltpu.VMEM((NREL, D), jnp.bfloat16),
            pltpu.VMEM((NP,), jnp.float32),
            pltpu.VMEM_SHARED((NP, D), jnp.float32),
            pltpu.SemaphoreType.DMA,
            pltpu.SemaphoreType.DMA,
            pltpu.SemaphoreType.DMA,
            pltpu.SemaphoreType.DMA,
            pltpu.SemaphoreType.DMA,
        ],
        compiler_params=pltpu.CompilerParams(
            needs_layout_passes=False, use_tc_tiling_on_sc=False),
    )(xtab, atabf, idx3, zcol, z128)


_BLK1 = 2000


def _mm1_body(x0_ref, w_ref, aself_ref, x1_ref, ui_ref):
    x0 = x0_ref[...]
    w = w_ref[0]
    y = jnp.dot(x0, w, preferred_element_type=jnp.float32)
    a = aself_ref[0]
    alpha = jnp.sum(y * y * a, axis=-1)
    alpha = jnp.where(alpha > 0, alpha, 0.2 * alpha)
    ee = jnp.exp(alpha)[:, None]
    x1_ref[0] = y
    ui_ref[0, :, :D] = ee * y
    lane16 = (lax.broadcasted_iota(jnp.int32, (1, 16), 1) == 0)
    ui_ref[0, :, D:UW] = ee * lane16.astype(jnp.float32)


def _mm1(x0, W1r, a1self):
    nb = N // _BLK1
    return pl.pallas_call(
        _mm1_body,
        grid=(H1, nb),
        in_specs=[
            pl.BlockSpec((_BLK1, D), lambda h, i: (i, 0)),
            pl.BlockSpec((1, D, D), lambda h, i: (h, 0, 0)),
            pl.BlockSpec((1, 1, D), lambda h, i: (h, 0, 0)),
        ],
        out_specs=[
            pl.BlockSpec((1, _BLK1, D), lambda h, i: (h, i, 0)),
            pl.BlockSpec((1, _BLK1, UW), lambda h, i: (h, i, 0)),
        ],
        out_shape=[
            jax.ShapeDtypeStruct((H1, N, D), jnp.float32),
            jax.ShapeDtypeStruct((H1, N, UW), jnp.float32),
        ],
    )(x0, W1r, a1self)


_BLK2 = 400


def _mid_body(ua_ref, ub_ref, dn_ref, ui_ref, b1_ref, w2_ref, a2_ref,
              x2_ref, ui2_ref):
    x2 = jnp.zeros((_BLK2, D), dtype=jnp.float32)
    for h in range(H1):
        num = ua_ref[h] + ub_ref[h] + ui_ref[h][:, :D]
        den = jnp.sum(dn_ref[h], axis=-1, keepdims=True) \
            + ui_ref[h][:, D:D + 1]
        o = num / (den + 1e-16) + b1_ref[h][None, :]
        hh = jnp.where(o > 0, o, 0.01 * o)
        x2 = x2 + jnp.dot(hh, w2_ref[h], preferred_element_type=jnp.float32)
    a2 = a2_ref[...]
    alpha = jnp.sum(x2 * x2 * a2, axis=-1)
    alpha = jnp.where(alpha > 0, alpha, 0.2 * alpha)
    ee = jnp.exp(alpha)[:, None]
    x2_ref[...] = x2
    ui2_ref[:, :D] = ee * x2
    lane16 = (lax.broadcasted_iota(jnp.int32, (1, 16), 1) == 0)
    ui2_ref[:, D:UW] = ee * lane16.astype(jnp.float32)


def _mid(Ua, Ub, dn1t, uinit1, b1r, W2r, a2self):
    nb = N // _BLK2
    return pl.pallas_call(
        _mid_body,
        grid=(nb,),
        in_specs=[
            pl.BlockSpec((H1, _BLK2, D), lambda i: (0, i, 0)),
            pl.BlockSpec((H1, _BLK2, D), lambda i: (0, i, 0)),
            pl.BlockSpec((H1, _BLK2, NC * NS), lambda i: (0, i, 0)),
            pl.BlockSpec((H1, _BLK2, UW), lambda i: (0, i, 0)),
            pl.BlockSpec((H1, D), lambda i: (0, 0)),
            pl.BlockSpec((H1, D, D), lambda i: (0, 0, 0)),
            pl.BlockSpec((1, D), lambda i: (0, 0)),
        ],
        out_specs=[
            pl.BlockSpec((_BLK2, D), lambda i: (i, 0)),
            pl.BlockSpec((_BLK2, UW), lambda i: (i, 0)),
        ],
        out_shape=[
            jax.ShapeDtypeStruct((N, D), jnp.float32),
            jax.ShapeDtypeStruct((N, UW), jnp.float32),
        ],
    )(Ua, Ub, dn1t, uinit1, b1r, W2r, a2self)


_BLK3 = 2000


def _fin_body(ua_ref, ub_ref, dn_ref, ui_ref, b2_ref, out_ref):
    num = ua_ref[...] + ub_ref[...] + ui_ref[:, :D]
    den = jnp.sum(dn_ref[...], axis=-1, keepdims=True) + ui_ref[:, D:D + 1]
    out_ref[...] = num / (den + 1e-16) + b2_ref[...]


def _fin(Ua, Ub, dn2t, uinit2, b2):
    nb = N // _BLK3
    return pl.pallas_call(
        _fin_body,
        grid=(nb,),
        in_specs=[
            pl.BlockSpec((_BLK3, D), lambda i: (i, 0)),
            pl.BlockSpec((_BLK3, D), lambda i: (i, 0)),
            pl.BlockSpec((_BLK3, NC * NS), lambda i: (i, 0)),
            pl.BlockSpec((_BLK3, UW), lambda i: (i, 0)),
            pl.BlockSpec((1, D), lambda i: (0, 0)),
        ],
        out_specs=pl.BlockSpec((_BLK3, D), lambda i: (i, 0)),
        out_shape=jax.ShapeDtypeStruct((N, D), jnp.float32),
    )(Ua, Ub, dn2t, uinit2, b2)


def kernel(entity, edge_index, edge_attr, emb_table, W1, att1, b1, W2, att2,
           b2):
    src = edge_index[0].reshape(NC * NS, NCHUNK, CHUNK)
    dst = edge_index[1].reshape(NC * NS, NCHUNK, CHUNK)
    ea = edge_attr.reshape(NC * NS, NCHUNK, CHUNK)
    idx3 = jnp.stack([src, dst, ea], axis=2).reshape(
        NC * NS * NCHUNK, 3, CHUNK)
    zcol = jnp.zeros((NP,), dtype=jnp.float32)
    z128 = jnp.zeros((NP, D), dtype=jnp.float32)

    def interleave_bf16(a):
        R = a.shape[0]
        return a.reshape(R, 4, 2, 16).transpose(0, 1, 3, 2).reshape(
            R, D).astype(jnp.bfloat16)

    idx_pad = jnp.pad(entity, (0, _EMB_B - N))
    x0 = _emb_gather(emb_table, idx_pad)[:N]

    W1r = W1.reshape(D, H1, D).transpose(1, 0, 2)
    a1self = att1[NREL - 1].reshape(H1, 1, D)
    x1h, uinit1 = _mm1(x0, W1r, a1self)

    att1h = att1.transpose(1, 0, 2)
    res1 = [_edge_pass(x1h[h], interleave_bf16(att1h[h]), idx3, zcol, z128)
            for h in range(H1)]
    Ua = jnp.stack([u[:N] for u, _ in res1])
    Ub = jnp.stack([u[NP:NP + N] for u, _ in res1])
    dn1t = jnp.stack([dn.reshape(NC * NS, NP)[:, :N].T for _, dn in res1])

    b1r = b1.reshape(H1, D)
    W2r = W2.reshape(H1, D, D)
    a2self = att2[NREL - 1]
    x2, uinit2 = _mid(Ua, Ub, dn1t, uinit1, b1r, W2r, a2self)

    U2, dn2 = _edge_pass(x2, interleave_bf16(att2[:, 0, :]), idx3, zcol, z128)
    dn2t = dn2.reshape(NC * NS, NP)[:, :N].T
    out = _fin(U2[:N], U2[NP:NP + N], dn2t, uinit2, b2.reshape(1, D))
    return out
